# baseline (speedup 1.0000x reference)
import functools

import jax
import jax.numpy as jnp
from jax import lax
from jax.experimental import pallas as pl
from jax.experimental.pallas import tpu as pltpu
from jax.experimental.pallas import tpu_sc as plsc

_N = 10000
_D = 128
_E = 320000
_NC = 2
_NS = 16
_NW = _NC * _NS
_NP = 10112
_RPT = _NP // _NS
_C = 64
_EP = 327680
_CPW = _EP // (_C * _NW)
_BC = 40

_PROP_STEP = 8
_ALP = 0.5

_mesh = plsc.VectorSubcoreMesh(core_axis_name="c", subcore_axis_name="s")


@functools.partial(
    pl.kernel,
    out_type=(
        jax.ShapeDtypeStruct((_NP, _D), jnp.float32),
        jax.ShapeDtypeStruct((_NP, _D), jnp.float32),
    ),
    mesh=_mesh,
    scratch_types=[
        pltpu.VMEM((_BC, _C), jnp.int32),
        pltpu.VMEM((_BC, _C), jnp.int32),
        [pltpu.VMEM((_C, _D), jnp.float32) for _ in range(4)],
        [pltpu.SemaphoreType.DMA for _ in range(4)],
        [pltpu.SemaphoreType.DMA for _ in range(4)],
        pltpu.VMEM_SHARED((_NP, _D), jnp.float32),
    ],
)
def _spmm_step(z_hbm, src_hbm, dst_hbm, zrows_hbm, p0_hbm, p1_hbm,
               sidx, didx, rows, semg, sems, acc):
    c = lax.axis_index("c")
    s = lax.axis_index("s")
    w = c * _NS + s
    base_row = s * _RPT

    pltpu.sync_copy(zrows_hbm, rows[0])
    for k in range(_RPT // _C):
        pltpu.sync_copy(rows[0], acc.at[pl.ds(base_row + k * _C, _C)])
    rem = _RPT % _C
    if rem:
        pltpu.sync_copy(rows[0].at[pl.ds(0, rem)],
                        acc.at[pl.ds(base_row + (_RPT // _C) * _C, rem)])
    plsc.subcore_barrier()

    def gather(l, j):
        pltpu.async_copy(z_hbm.at[sidx.at[l]], rows[j], semg[j])

    def wait_gather(l, j):
        pltpu.make_async_copy(z_hbm.at[sidx.at[l]], rows[j], semg[j]).wait()

    def scatter(l, j):
        pltpu.async_copy(rows[j], acc.at[didx.at[l]], sems[j], add=True)

    def wait_scatter(l, j):
        pltpu.make_async_copy(rows[j], acc.at[didx.at[l]], sems[j]).wait()

    for b in range(_CPW // _BC):
        pltpu.sync_copy(src_hbm.at[pl.ds(w * _CPW + b * _BC, _BC)], sidx)
        pltpu.sync_copy(dst_hbm.at[pl.ds(w * _CPW + b * _BC, _BC)], didx)
        gather(0, 0)
        gather(1, 1)

        def body(h, _):
            for j in range(4):
                l = 4 * h + j
                jn = (j + 2) % 4
                wait_gather(l, j)
                scatter(l, j)

                @pl.when(l >= 2)
                def _():
                    wait_scatter(l - 2, jn)

                @pl.when(l + 2 < _BC)
                def _():
                    gather(l + 2, jn)
            return 0

        lax.fori_loop(0, _BC // 4, body, 0)
        wait_scatter(_BC - 2, (_BC - 2) % 4)
        wait_scatter(_BC - 1, (_BC - 1) % 4)
    plsc.subcore_barrier()

    @pl.when(c == 0)
    def _():
        pltpu.sync_copy(acc.at[pl.ds(base_row, _RPT)],
                        p0_hbm.at[pl.ds(base_row, _RPT)])

    @pl.when(c == 1)
    def _():
        pltpu.sync_copy(acc.at[pl.ds(base_row, _RPT)],
                        p1_hbm.at[pl.ds(base_row, _RPT)])


@functools.partial(
    pl.kernel,
    out_type=(
        jax.ShapeDtypeStruct((_NP, 16), jnp.float32),
        jax.ShapeDtypeStruct((_NP, 16), jnp.float32),
    ),
    mesh=_mesh,
    scratch_types=[
        pltpu.VMEM((_EP // (128 * _NW), 128), jnp.int32),
        pltpu.VMEM((128, 16), jnp.float32),
        pltpu.VMEM((128, 16), jnp.float32),
        pltpu.VMEM_SHARED((_NP, 16), jnp.float32),
    ],
)
def _degree(dst_hbm, d0_hbm, d1_hbm, didx, ones, zbuf, acc):
    c = lax.axis_index("c")
    s = lax.axis_index("s")
    w = c * _NS + s
    base_row = s * _RPT
    cpw = _EP // (128 * _NW)

    pltpu.sync_copy(dst_hbm.at[pl.ds(w * cpw, cpw)], didx)

    ov = jnp.ones((16,), jnp.float32)
    zv = jnp.zeros((16,), jnp.float32)

    def fill(k, _):
        ones[k, pl.ds(0, 16)] = ov
        zbuf[k, pl.ds(0, 16)] = zv
        return 0

    lax.fori_loop(0, 128, fill, 0)

    for k in range(_RPT // 128):
        pltpu.sync_copy(zbuf, acc.at[pl.ds(base_row + k * 128, 128)])
    rem = _RPT % 128
    if rem:
        pltpu.sync_copy(zbuf.at[pl.ds(0, rem)],
                        acc.at[pl.ds(base_row + (_RPT // 128) * 128, rem)])
    plsc.subcore_barrier()

    def chunk(g, _):
        pltpu.sync_copy(ones, acc.at[didx.at[g]], add=True)
        return 0

    lax.fori_loop(0, cpw, chunk, 0)
    plsc.subcore_barrier()

    @pl.when(c == 0)
    def _():
        pltpu.sync_copy(acc.at[pl.ds(base_row, _RPT)],
                        d0_hbm.at[pl.ds(base_row, _RPT)])

    @pl.when(c == 1)
    def _():
        pltpu.sync_copy(acc.at[pl.ds(base_row, _RPT)],
                        d1_hbm.at[pl.ds(base_row, _RPT)])


def kernel(x, edge_index):
    src = edge_index[0].astype(jnp.int32)
    dst = edge_index[1].astype(jnp.int32)

    npad = _EP - _E
    pad_src = (jnp.arange(npad, dtype=jnp.int32) * 7919) % _N
    pad_dst = _N + (jnp.arange(npad, dtype=jnp.int32) % (_NP - _N))
    src_flat = jnp.concatenate([src, pad_src])
    dst_flat = jnp.concatenate([dst, pad_dst])
    src_p = src_flat.reshape(_NW * _CPW, _C)
    dst_p = dst_flat.reshape(_NW * _CPW, _C)

    d0, d1 = _degree(dst_flat.reshape(-1, 128))
    deg = (d0 + d1)[:, 0]

    valid = jnp.arange(_NP) < _N
    inv = jnp.where(deg > 0, 1.0 / jnp.where(deg > 0, deg, 1.0), jnp.inf)
    rsq = jnp.where(valid & (deg > 0), lax.rsqrt(jnp.where(deg > 0, deg, 1.0)), 0.0)
    c1 = jnp.where(valid, 1.0 - _ALP * inv - _ALP, 0.0)[:, None]
    c2 = jnp.where(valid, _ALP * inv, 0.0)[:, None]
    rsq = rsq[:, None]

    zrows = jnp.zeros((_C, _D), jnp.float32)
    y0 = jnp.pad(x, ((0, _NP - _N), (0, 0)))
    y = y0
    z = rsq * y
    for _ in range(_PROP_STEP):
        p0, p1 = _spmm_step(z, src_p, dst_p, zrows)
        y = c1 * y + c2 * y0 + (_ALP * rsq) * (p0 + p1)
        z = rsq * y
    return y[:_N]


# speedup vs baseline: 1.1238x; 1.1238x over previous
import functools

import jax
import jax.numpy as jnp
from jax import lax
from jax.experimental import pallas as pl
from jax.experimental.pallas import tpu as pltpu
from jax.experimental.pallas import tpu_sc as plsc

_N = 10000
_D = 128
_E = 320000
_NC = 2
_NS = 16
_NW = _NC * _NS
_NP = 10112
_RPT = _NP // _NS
_C = 128
_EP = 327680
_CPW = _EP // (_C * _NW)
_BC = 40

_PROP_STEP = 8
_ALP = 0.5

_mesh = plsc.VectorSubcoreMesh(core_axis_name="c", subcore_axis_name="s")


@functools.partial(
    pl.kernel,
    out_type=(
        jax.ShapeDtypeStruct((_NP, _D), jnp.float32),
        jax.ShapeDtypeStruct((_NP, _D), jnp.float32),
    ),
    mesh=_mesh,
    scratch_types=[
        pltpu.VMEM((_BC, _C), jnp.int32),
        pltpu.VMEM((_BC, _C), jnp.int32),
        [pltpu.VMEM((_C, _D), jnp.float32) for _ in range(2)],
        [pltpu.SemaphoreType.DMA for _ in range(2)],
        pltpu.VMEM_SHARED((_NP, _D), jnp.float32),
    ],
)
def _spmm_step(z_hbm, src_hbm, dst_hbm, zrows_hbm, p0_hbm, p1_hbm,
               sidx, didx, rows, semg, acc):
    c = lax.axis_index("c")
    s = lax.axis_index("s")
    w = c * _NS + s
    base_row = s * _RPT

    pltpu.sync_copy(zrows_hbm, rows[0])
    for k in range(_RPT // _C):
        pltpu.sync_copy(rows[0], acc.at[pl.ds(base_row + k * _C, _C)])
    rem = _RPT % _C
    if rem:
        pltpu.sync_copy(rows[0].at[pl.ds(0, rem)],
                        acc.at[pl.ds(base_row + (_RPT // _C) * _C, rem)])
    plsc.subcore_barrier()

    for b in range(_CPW // _BC):
        pltpu.sync_copy(src_hbm.at[pl.ds(w * _CPW + b * _BC, _BC)], sidx)
        pltpu.sync_copy(dst_hbm.at[pl.ds(w * _CPW + b * _BC, _BC)], didx)
        pltpu.async_copy(z_hbm.at[sidx.at[0]], rows[0], semg[0])

        def body(h, _):
            g0 = 2 * h
            pltpu.async_copy(z_hbm.at[sidx.at[g0 + 1]], rows[1], semg[1])
            pltpu.make_async_copy(z_hbm.at[sidx.at[g0]], rows[0], semg[0]).wait()
            pltpu.sync_copy(rows[0], acc.at[didx.at[g0]], add=True)

            @pl.when(h < _BC // 2 - 1)
            def _():
                pltpu.async_copy(z_hbm.at[sidx.at[g0 + 2]], rows[0], semg[0])

            pltpu.make_async_copy(z_hbm.at[sidx.at[g0 + 1]], rows[1], semg[1]).wait()
            pltpu.sync_copy(rows[1], acc.at[didx.at[g0 + 1]], add=True)
            return 0

        lax.fori_loop(0, _BC // 2, body, 0)
    plsc.subcore_barrier()

    @pl.when(c == 0)
    def _():
        pltpu.sync_copy(acc.at[pl.ds(base_row, _RPT)],
                        p0_hbm.at[pl.ds(base_row, _RPT)])

    @pl.when(c == 1)
    def _():
        pltpu.sync_copy(acc.at[pl.ds(base_row, _RPT)],
                        p1_hbm.at[pl.ds(base_row, _RPT)])


@functools.partial(
    pl.kernel,
    out_type=(
        jax.ShapeDtypeStruct((_NP, 16), jnp.float32),
        jax.ShapeDtypeStruct((_NP, 16), jnp.float32),
    ),
    mesh=_mesh,
    scratch_types=[
        pltpu.VMEM((_EP // (128 * _NW), 128), jnp.int32),
        pltpu.VMEM((128, 16), jnp.float32),
        pltpu.VMEM((128, 16), jnp.float32),
        pltpu.VMEM_SHARED((_NP, 16), jnp.float32),
    ],
)
def _degree(dst_hbm, d0_hbm, d1_hbm, didx, ones, zbuf, acc):
    c = lax.axis_index("c")
    s = lax.axis_index("s")
    w = c * _NS + s
    base_row = s * _RPT
    cpw = _EP // (128 * _NW)

    pltpu.sync_copy(dst_hbm.at[pl.ds(w * cpw, cpw)], didx)

    ov = jnp.ones((16,), jnp.float32)
    zv = jnp.zeros((16,), jnp.float32)

    def fill(k, _):
        ones[k, pl.ds(0, 16)] = ov
        zbuf[k, pl.ds(0, 16)] = zv
        return 0

    lax.fori_loop(0, 128, fill, 0)

    for k in range(_RPT // 128):
        pltpu.sync_copy(zbuf, acc.at[pl.ds(base_row + k * 128, 128)])
    rem = _RPT % 128
    if rem:
        pltpu.sync_copy(zbuf.at[pl.ds(0, rem)],
                        acc.at[pl.ds(base_row + (_RPT // 128) * 128, rem)])
    plsc.subcore_barrier()

    def chunk(g, _):
        pltpu.sync_copy(ones, acc.at[didx.at[g]], add=True)
        return 0

    lax.fori_loop(0, cpw, chunk, 0)
    plsc.subcore_barrier()

    @pl.when(c == 0)
    def _():
        pltpu.sync_copy(acc.at[pl.ds(base_row, _RPT)],
                        d0_hbm.at[pl.ds(base_row, _RPT)])

    @pl.when(c == 1)
    def _():
        pltpu.sync_copy(acc.at[pl.ds(base_row, _RPT)],
                        d1_hbm.at[pl.ds(base_row, _RPT)])


def kernel(x, edge_index):
    src = edge_index[0].astype(jnp.int32)
    dst = edge_index[1].astype(jnp.int32)

    npad = _EP - _E
    pad_src = (jnp.arange(npad, dtype=jnp.int32) * 7919) % _N
    pad_dst = _N + (jnp.arange(npad, dtype=jnp.int32) % (_NP - _N))
    src_flat = jnp.concatenate([src, pad_src])
    dst_flat = jnp.concatenate([dst, pad_dst])
    src_p = src_flat.reshape(_NW * _CPW, _C)
    dst_p = dst_flat.reshape(_NW * _CPW, _C)

    d0, d1 = _degree(dst_flat.reshape(-1, 128))
    deg = (d0 + d1)[:, 0]

    valid = jnp.arange(_NP) < _N
    inv = jnp.where(deg > 0, 1.0 / jnp.where(deg > 0, deg, 1.0), jnp.inf)
    rsq = jnp.where(valid & (deg > 0), lax.rsqrt(jnp.where(deg > 0, deg, 1.0)), 0.0)
    c1 = jnp.where(valid, 1.0 - _ALP * inv - _ALP, 0.0)[:, None]
    c2 = jnp.where(valid, _ALP * inv, 0.0)[:, None]
    rsq = rsq[:, None]

    zrows = jnp.zeros((_C, _D), jnp.float32)
    y0 = jnp.pad(x, ((0, _NP - _N), (0, 0)))
    y = y0
    z = rsq * y
    for _ in range(_PROP_STEP):
        p0, p1 = _spmm_step(z, src_p, dst_p, zrows)
        y = c1 * y + c2 * y0 + (_ALP * rsq) * (p0 + p1)
        z = rsq * y
    return y[:_N]


# speedup vs baseline: 1.1659x; 1.0374x over previous
import functools

import jax
import jax.numpy as jnp
from jax import lax
from jax.experimental import pallas as pl
from jax.experimental.pallas import tpu as pltpu
from jax.experimental.pallas import tpu_sc as plsc

_N = 10000
_D = 128
_E = 320000
_NC = 2
_NS = 16
_NW = _NC * _NS
_NP = 10112
_RPT = _NP // _NS
_C = 64
_EP = 327680
_CPW = _EP // (_C * _NW)
_BC = 40
_NB = 3

_PROP_STEP = 8
_ALP = 0.5

_mesh = plsc.VectorSubcoreMesh(core_axis_name="c", subcore_axis_name="s")


@functools.partial(
    pl.kernel,
    out_type=(
        jax.ShapeDtypeStruct((_NP, _D), jnp.float32),
        jax.ShapeDtypeStruct((_NP, _D), jnp.float32),
    ),
    mesh=_mesh,
    scratch_types=[
        [pltpu.VMEM((_BC, _C), jnp.int32) for _ in range(2)],
        [pltpu.VMEM((_BC, _C), jnp.int32) for _ in range(2)],
        [pltpu.VMEM((_C, _D), jnp.float32) for _ in range(_NB)],
        [pltpu.SemaphoreType.DMA for _ in range(_NB)],
        [pltpu.SemaphoreType.DMA for _ in range(2)],
        pltpu.VMEM_SHARED((_NP, _D), jnp.float32),
    ],
)
def _spmm_step(z_hbm, src_hbm, dst_hbm, zrows_hbm, p0_hbm, p1_hbm,
               sblk, dblk, rows, semg, semi, acc):
    c = lax.axis_index("c")
    s = lax.axis_index("s")
    w = c * _NS + s
    base_row = s * _RPT

    pltpu.sync_copy(zrows_hbm, rows[0])
    for k in range(_RPT // _C):
        pltpu.sync_copy(rows[0], acc.at[pl.ds(base_row + k * _C, _C)])
    rem = _RPT % _C
    if rem:
        pltpu.sync_copy(rows[0].at[pl.ds(0, rem)],
                        acc.at[pl.ds(base_row + (_RPT // _C) * _C, rem)])
    plsc.subcore_barrier()

    nblk = _CPW // _BC
    for b in range(nblk):
        pr = b % 2
        if b == 0:
            pltpu.sync_copy(src_hbm.at[pl.ds(w * _CPW, _BC)], sblk[0])
            pltpu.sync_copy(dst_hbm.at[pl.ds(w * _CPW, _BC)], dblk[0])
        if b + 1 < nblk:
            pn = (b + 1) % 2
            pltpu.async_copy(src_hbm.at[pl.ds(w * _CPW + (b + 1) * _BC, _BC)],
                             sblk[pn], semi[pn])
            pltpu.async_copy(dst_hbm.at[pl.ds(w * _CPW + (b + 1) * _BC, _BC)],
                             dblk[pn], semi[pn])
        if b > 0:
            pltpu.make_async_copy(
                src_hbm.at[pl.ds(w * _CPW + b * _BC, _BC)], sblk[pr],
                semi[pr]).wait()
            pltpu.make_async_copy(
                dst_hbm.at[pl.ds(w * _CPW + b * _BC, _BC)], dblk[pr],
                semi[pr]).wait()
        sidx = sblk[pr]
        didx = dblk[pr]
        for p in range(_NB - 1):
            pltpu.async_copy(z_hbm.at[sidx.at[p]], rows[p], semg[p])

        def body(i, _):
            for j in range(_NB):
                g = _NB * i + j
                jn = (j + _NB - 1) % _NB

                @pl.when(g + _NB - 1 < _BC)
                def _():
                    pltpu.async_copy(z_hbm.at[sidx.at[g + _NB - 1]],
                                     rows[jn], semg[jn])

                pltpu.make_async_copy(z_hbm.at[sidx.at[g]], rows[j],
                                      semg[j]).wait()
                pltpu.sync_copy(rows[j], acc.at[didx.at[g]], add=True)
            return 0

        lax.fori_loop(0, _BC // _NB, body, 0)
        for t in range((_BC // _NB) * _NB, _BC):
            j = t % _NB
            pltpu.make_async_copy(z_hbm.at[sidx.at[t]], rows[j],
                                  semg[j]).wait()
            pltpu.sync_copy(rows[j], acc.at[didx.at[t]], add=True)
    plsc.subcore_barrier()

    @pl.when(c == 0)
    def _():
        pltpu.sync_copy(acc.at[pl.ds(base_row, _RPT)],
                        p0_hbm.at[pl.ds(base_row, _RPT)])

    @pl.when(c == 1)
    def _():
        pltpu.sync_copy(acc.at[pl.ds(base_row, _RPT)],
                        p1_hbm.at[pl.ds(base_row, _RPT)])


@functools.partial(
    pl.kernel,
    out_type=(
        jax.ShapeDtypeStruct((_NP, 16), jnp.float32),
        jax.ShapeDtypeStruct((_NP, 16), jnp.float32),
    ),
    mesh=_mesh,
    scratch_types=[
        pltpu.VMEM((_EP // (128 * _NW), 128), jnp.int32),
        pltpu.VMEM((128, 16), jnp.float32),
        pltpu.VMEM((128, 16), jnp.float32),
        pltpu.VMEM_SHARED((_NP, 16), jnp.float32),
    ],
)
def _degree(dst_hbm, d0_hbm, d1_hbm, didx, ones, zbuf, acc):
    c = lax.axis_index("c")
    s = lax.axis_index("s")
    w = c * _NS + s
    base_row = s * _RPT
    cpw = _EP // (128 * _NW)

    pltpu.sync_copy(dst_hbm.at[pl.ds(w * cpw, cpw)], didx)

    ov = jnp.ones((16,), jnp.float32)
    zv = jnp.zeros((16,), jnp.float32)

    def fill(k, _):
        ones[k, pl.ds(0, 16)] = ov
        zbuf[k, pl.ds(0, 16)] = zv
        return 0

    lax.fori_loop(0, 128, fill, 0)

    for k in range(_RPT // 128):
        pltpu.sync_copy(zbuf, acc.at[pl.ds(base_row + k * 128, 128)])
    rem = _RPT % 128
    if rem:
        pltpu.sync_copy(zbuf.at[pl.ds(0, rem)],
                        acc.at[pl.ds(base_row + (_RPT // 128) * 128, rem)])
    plsc.subcore_barrier()

    def chunk(g, _):
        pltpu.sync_copy(ones, acc.at[didx.at[g]], add=True)
        return 0

    lax.fori_loop(0, cpw, chunk, 0)
    plsc.subcore_barrier()

    @pl.when(c == 0)
    def _():
        pltpu.sync_copy(acc.at[pl.ds(base_row, _RPT)],
                        d0_hbm.at[pl.ds(base_row, _RPT)])

    @pl.when(c == 1)
    def _():
        pltpu.sync_copy(acc.at[pl.ds(base_row, _RPT)],
                        d1_hbm.at[pl.ds(base_row, _RPT)])


def kernel(x, edge_index):
    src = edge_index[0].astype(jnp.int32)
    dst = edge_index[1].astype(jnp.int32)

    npad = _EP - _E
    pad_src = (jnp.arange(npad, dtype=jnp.int32) * 7919) % _N
    pad_dst = _N + (jnp.arange(npad, dtype=jnp.int32) % (_NP - _N))
    src_flat = jnp.concatenate([src, pad_src])
    dst_flat = jnp.concatenate([dst, pad_dst])
    src_p = src_flat.reshape(_NW * _CPW, _C)
    dst_p = dst_flat.reshape(_NW * _CPW, _C)

    d0, d1 = _degree(dst_flat.reshape(-1, 128))
    deg = (d0 + d1)[:, 0]

    valid = jnp.arange(_NP) < _N
    inv = jnp.where(deg > 0, 1.0 / jnp.where(deg > 0, deg, 1.0), jnp.inf)
    rsq = jnp.where(valid & (deg > 0), lax.rsqrt(jnp.where(deg > 0, deg, 1.0)), 0.0)
    c1 = jnp.where(valid, 1.0 - _ALP * inv - _ALP, 0.0)[:, None]
    c2 = jnp.where(valid, _ALP * inv, 0.0)[:, None]
    rsq = rsq[:, None]

    zrows = jnp.zeros((_C, _D), jnp.float32)
    y0 = jnp.pad(x, ((0, _NP - _N), (0, 0)))
    y = y0
    z = rsq * y
    for _ in range(_PROP_STEP):
        p0, p1 = _spmm_step(z, src_p, dst_p, zrows)
        y = c1 * y + c2 * y0 + (_ALP * rsq) * (p0 + p1)
        z = rsq * y
    return y[:_N]


# speedup vs baseline: 1.1854x; 1.0167x over previous
import functools

import jax
import jax.numpy as jnp
from jax import lax
from jax.experimental import pallas as pl
from jax.experimental.pallas import tpu as pltpu
from jax.experimental.pallas import tpu_sc as plsc

_N = 10000
_D = 128
_E = 320000
_NC = 2
_NS = 16
_NW = _NC * _NS
_NP = 10112
_RPT = _NP // _NS
_C = 64
_EP = 327680
_CPW = _EP // (_C * _NW)
_BC = 80
_NB = 3

_PROP_STEP = 8
_ALP = 0.5

_mesh = plsc.VectorSubcoreMesh(core_axis_name="c", subcore_axis_name="s")


@functools.partial(
    pl.kernel,
    out_type=(
        jax.ShapeDtypeStruct((_NP, _D), jnp.float32),
        jax.ShapeDtypeStruct((_NP, _D), jnp.float32),
    ),
    mesh=_mesh,
    scratch_types=[
        pltpu.VMEM((_BC, _C), jnp.int32),
        pltpu.VMEM((_BC, _C), jnp.int32),
        [pltpu.VMEM((_C, _D), jnp.float32) for _ in range(_NB)],
        [pltpu.SemaphoreType.DMA for _ in range(_NB)],
        pltpu.VMEM_SHARED((_NP, _D), jnp.float32),
    ],
)
def _spmm_step(z_hbm, src_hbm, dst_hbm, zrows_hbm, p0_hbm, p1_hbm,
               sidx, didx, rows, semg, acc):
    c = lax.axis_index("c")
    s = lax.axis_index("s")
    w = c * _NS + s
    base_row = s * _RPT

    pltpu.sync_copy(zrows_hbm, rows[0])
    for k in range(_RPT // _C):
        pltpu.sync_copy(rows[0], acc.at[pl.ds(base_row + k * _C, _C)])
    rem = _RPT % _C
    if rem:
        pltpu.sync_copy(rows[0].at[pl.ds(0, rem)],
                        acc.at[pl.ds(base_row + (_RPT // _C) * _C, rem)])
    plsc.subcore_barrier()

    for b in range(_CPW // _BC):
        pltpu.sync_copy(src_hbm.at[pl.ds(w * _CPW + b * _BC, _BC)], sidx)
        pltpu.sync_copy(dst_hbm.at[pl.ds(w * _CPW + b * _BC, _BC)], didx)
        for p in range(_NB - 1):
            pltpu.async_copy(z_hbm.at[sidx.at[p]], rows[p], semg[p])

        def body(i, _):
            for j in range(_NB):
                g = _NB * i + j
                jn = (j + _NB - 1) % _NB

                @pl.when(g + _NB - 1 < _BC)
                def _():
                    pltpu.async_copy(z_hbm.at[sidx.at[g + _NB - 1]],
                                     rows[jn], semg[jn])

                pltpu.make_async_copy(z_hbm.at[sidx.at[g]], rows[j],
                                      semg[j]).wait()
                pltpu.sync_copy(rows[j], acc.at[didx.at[g]], add=True)
            return 0

        lax.fori_loop(0, _BC // _NB, body, 0)
        for t in range((_BC // _NB) * _NB, _BC):
            j = t % _NB
            pltpu.make_async_copy(z_hbm.at[sidx.at[t]], rows[j],
                                  semg[j]).wait()
            pltpu.sync_copy(rows[j], acc.at[didx.at[t]], add=True)
    plsc.subcore_barrier()

    @pl.when(c == 0)
    def _():
        pltpu.sync_copy(acc.at[pl.ds(base_row, _RPT)],
                        p0_hbm.at[pl.ds(base_row, _RPT)])

    @pl.when(c == 1)
    def _():
        pltpu.sync_copy(acc.at[pl.ds(base_row, _RPT)],
                        p1_hbm.at[pl.ds(base_row, _RPT)])


@functools.partial(
    pl.kernel,
    out_type=(
        jax.ShapeDtypeStruct((_NP, 16), jnp.float32),
        jax.ShapeDtypeStruct((_NP, 16), jnp.float32),
    ),
    mesh=_mesh,
    scratch_types=[
        pltpu.VMEM((_EP // (128 * _NW), 128), jnp.int32),
        pltpu.VMEM((128, 16), jnp.float32),
        pltpu.VMEM((128, 16), jnp.float32),
        pltpu.VMEM_SHARED((_NP, 16), jnp.float32),
    ],
)
def _degree(dst_hbm, d0_hbm, d1_hbm, didx, ones, zbuf, acc):
    c = lax.axis_index("c")
    s = lax.axis_index("s")
    w = c * _NS + s
    base_row = s * _RPT
    cpw = _EP // (128 * _NW)

    pltpu.sync_copy(dst_hbm.at[pl.ds(w * cpw, cpw)], didx)

    ov = jnp.ones((16,), jnp.float32)
    zv = jnp.zeros((16,), jnp.float32)

    def fill(k, _):
        ones[k, pl.ds(0, 16)] = ov
        zbuf[k, pl.ds(0, 16)] = zv
        return 0

    lax.fori_loop(0, 128, fill, 0)

    for k in range(_RPT // 128):
        pltpu.sync_copy(zbuf, acc.at[pl.ds(base_row + k * 128, 128)])
    rem = _RPT % 128
    if rem:
        pltpu.sync_copy(zbuf.at[pl.ds(0, rem)],
                        acc.at[pl.ds(base_row + (_RPT // 128) * 128, rem)])
    plsc.subcore_barrier()

    def chunk(g, _):
        pltpu.sync_copy(ones, acc.at[didx.at[g]], add=True)
        return 0

    lax.fori_loop(0, cpw, chunk, 0)
    plsc.subcore_barrier()

    @pl.when(c == 0)
    def _():
        pltpu.sync_copy(acc.at[pl.ds(base_row, _RPT)],
                        d0_hbm.at[pl.ds(base_row, _RPT)])

    @pl.when(c == 1)
    def _():
        pltpu.sync_copy(acc.at[pl.ds(base_row, _RPT)],
                        d1_hbm.at[pl.ds(base_row, _RPT)])


def kernel(x, edge_index):
    src = edge_index[0].astype(jnp.int32)
    dst = edge_index[1].astype(jnp.int32)

    npad = _EP - _E
    pad_src = (jnp.arange(npad, dtype=jnp.int32) * 7919) % _N
    pad_dst = _N + (jnp.arange(npad, dtype=jnp.int32) % (_NP - _N))
    src_flat = jnp.concatenate([src, pad_src])
    dst_flat = jnp.concatenate([dst, pad_dst])
    src_p = src_flat.reshape(_NW * _CPW, _C)
    dst_p = dst_flat.reshape(_NW * _CPW, _C)

    d0, d1 = _degree(dst_flat.reshape(-1, 128))
    deg = (d0 + d1)[:, 0]

    valid = jnp.arange(_NP) < _N
    inv = jnp.where(deg > 0, 1.0 / jnp.where(deg > 0, deg, 1.0), jnp.inf)
    rsq = jnp.where(valid & (deg > 0), lax.rsqrt(jnp.where(deg > 0, deg, 1.0)), 0.0)
    c1 = jnp.where(valid, 1.0 - _ALP * inv - _ALP, 0.0)[:, None]
    c2 = jnp.where(valid, _ALP * inv, 0.0)[:, None]
    rsq = rsq[:, None]

    zrows = jnp.zeros((_C, _D), jnp.float32)
    y0 = jnp.pad(x, ((0, _NP - _N), (0, 0)))
    y = y0
    z = rsq * y
    for _ in range(_PROP_STEP):
        p0, p1 = _spmm_step(z, src_p, dst_p, zrows)
        y = c1 * y + c2 * y0 + (_ALP * rsq) * (p0 + p1)
        z = rsq * y
    return y[:_N]
